# baseline (speedup 1.0000x reference)
import functools

import jax
import jax.numpy as jnp
from jax import lax
from jax.experimental import pallas as pl
from jax.experimental.pallas import tpu as pltpu
from jax.experimental.pallas import tpu_sc as plsc

N = 10000
D = 128
E = 320000
NW = 32
K = 50
CH = 200
N_PAD = 10240
RPT = N_PAD // 16

_mesh = plsc.VectorSubcoreMesh(core_axis_name="c", subcore_axis_name="s")


@functools.partial(
    pl.kernel,
    out_type=jax.ShapeDtypeStruct((4 * N_PAD,), jnp.float32),
    mesh=_mesh,
    scratch_types=[
        pltpu.VMEM((CH, K), jnp.int32),
        pltpu.VMEM((CH, K), jnp.int32),
        pltpu.VMEM((K,), jnp.float32),
        pltpu.VMEM_SHARED((N_PAD,), jnp.float32),
        pltpu.VMEM_SHARED((N_PAD,), jnp.float32),
        pltpu.SemaphoreType.DMA,
        pltpu.SemaphoreType.DMA,
    ],
)
def _degrees(src_hbm, dst_hbm, ones_hbm, zeros_hbm, out_hbm,
             src_v, dst_v, ones_v, acc_s, acc_d, sem_a, sem_b):
    c = lax.axis_index("c")
    s = lax.axis_index("s")
    w = c * 16 + s
    pltpu.sync_copy(src_hbm.at[w], src_v)
    pltpu.sync_copy(dst_hbm.at[w], dst_v)
    pltpu.sync_copy(ones_hbm, ones_v)
    pltpu.sync_copy(zeros_hbm, acc_s.at[pl.ds(s * RPT, RPT)])
    pltpu.sync_copy(zeros_hbm, acc_d.at[pl.ds(s * RPT, RPT)])
    plsc.subcore_barrier()

    DGRP = 8

    def gbody(g, _):
        cps = []
        for b in range(DGRP):
            j = g * DGRP + b
            cps.append(pltpu.async_copy(ones_v, acc_s.at[src_v.at[j]], sem_a,
                                        add=True))
            cps.append(pltpu.async_copy(ones_v, acc_d.at[dst_v.at[j]], sem_b,
                                        add=True))
        for cp in cps:
            cp.wait()
        return 0

    lax.fori_loop(0, CH // DGRP, gbody, 0)
    plsc.subcore_barrier()
    base = c * 2 * N_PAD
    pltpu.sync_copy(acc_s.at[pl.ds(s * RPT, RPT)],
                    out_hbm.at[pl.ds(base + s * RPT, RPT)])
    pltpu.sync_copy(acc_d.at[pl.ds(s * RPT, RPT)],
                    out_hbm.at[pl.ds(base + N_PAD + s * RPT, RPT)])


GRP = 8
NG = CH // GRP
NBUF = 6


@functools.partial(
    pl.kernel,
    out_type=jax.ShapeDtypeStruct((2 * N_PAD, D), jnp.float32),
    mesh=_mesh,
    scratch_types=[
        pltpu.VMEM((2, GRP, K), jnp.int32),
        pltpu.VMEM((2, GRP, K), jnp.int32),
        pltpu.VMEM((NBUF, K, D), jnp.float32),
        pltpu.VMEM_SHARED((N_PAD, D), jnp.float32),
        pltpu.SemaphoreType.DMA,
        pltpu.SemaphoreType.DMA,
        pltpu.SemaphoreType.DMA,
        pltpu.SemaphoreType.DMA,
        pltpu.SemaphoreType.DMA,
        pltpu.SemaphoreType.DMA,
        pltpu.SemaphoreType.DMA,
        pltpu.SemaphoreType.DMA,
        pltpu.SemaphoreType.DMA,
        pltpu.SemaphoreType.DMA,
        pltpu.SemaphoreType.DMA,
        pltpu.SemaphoreType.DMA,
        pltpu.SemaphoreType.DMA,
        pltpu.SemaphoreType.DMA,
    ],
)
def _edge_pass(t_hbm, src_hbm, dst_hbm, zeros_hbm, out_hbm,
               srcb, dstb, rows, acc,
               gsem0, gsem1, gsem2, gsem3, gsem4, gsem5,
               ssem0, ssem1, ssem2, ssem3, ssem4, ssem5,
               isem0, isem1):
    c = lax.axis_index("c")
    s = lax.axis_index("s")
    w = c * 16 + s
    my_src = src_hbm.at[w]
    my_dst = dst_hbm.at[w]
    pltpu.sync_copy(my_src.at[pl.ds(0, GRP)], srcb.at[0])
    pltpu.sync_copy(my_dst.at[pl.ds(0, GRP)], dstb.at[0])
    pltpu.sync_copy(zeros_hbm, acc.at[pl.ds(s * RPT, RPT)])
    plsc.subcore_barrier()

    gs = (gsem0, gsem1, gsem2, gsem3, gsem4, gsem5)
    ss = (ssem0, ssem1, ssem2, ssem3, ssem4, ssem5)
    ip = None
    if NG > 1:
        ip = (pltpu.async_copy(my_src.at[pl.ds(GRP, GRP)], srcb.at[1], isem0),
              pltpu.async_copy(my_dst.at[pl.ds(GRP, GRP)], dstb.at[1], isem1))

    pend_g = {}
    pend_s = {}
    waited_s = set()
    staged = 1

    for cn in range(min(NBUF, CH)):
        b = cn % NBUF
        pend_g[cn] = pltpu.async_copy(
            t_hbm.at[srcb.at[0].at[cn]], rows.at[b], gs[b])

    for cn in range(CH):
        b = cn % NBUF
        pend_g[cn].wait()
        nc = cn + NBUF - 1
        if cn >= 1 and nc < CH:
            g3 = nc // GRP
            if g3 >= staged:
                for cp in ip:
                    cp.wait()
                staged += 1
            pb = (cn - 1) % NBUF
            pend_s[cn - 1].wait()
            waited_s.add(cn - 1)
            pend_g[nc] = pltpu.async_copy(
                t_hbm.at[srcb.at[g3 % 2].at[nc % GRP]], rows.at[pb], gs[pb])
        pend_s[cn] = pltpu.async_copy(
            rows.at[b], acc.at[dstb.at[(cn // GRP) % 2].at[cn % GRP]],
            ss[b], add=True)
        if cn % GRP == 0 and cn > 0:
            gprev = cn // GRP - 1
            if gprev + 2 < NG:
                ip = (pltpu.async_copy(
                          my_src.at[pl.ds((gprev + 2) * GRP, GRP)],
                          srcb.at[gprev % 2], isem0),
                      pltpu.async_copy(
                          my_dst.at[pl.ds((gprev + 2) * GRP, GRP)],
                          dstb.at[gprev % 2], isem1))

    for cn in range(CH):
        if cn not in waited_s:
            pend_s[cn].wait()

    plsc.subcore_barrier()
    pltpu.sync_copy(
        acc.at[pl.ds(s * RPT, RPT)],
        out_hbm.at[pl.ds(c * N_PAD + s * RPT, RPT)],
    )


def _mm_body(x_ref, w_ref, o_ref):
    o_ref[...] = jnp.dot(x_ref[...], w_ref[...],
                         preferred_element_type=jnp.float32)


_matmul = pl.pallas_call(
    _mm_body, out_shape=jax.ShapeDtypeStruct((N_PAD, D), jnp.float32)
)


def _scale_norms_body(p_ref, degp_ref, t_ref, n_ref):
    x = degp_ref[...]
    ns = lax.rsqrt(jnp.maximum(x[0] + x[2], 1.0))
    nd = lax.rsqrt(jnp.maximum(x[1] + x[3], 1.0))
    n_ref[:, 0:1] = ns
    n_ref[:, 1:2] = nd
    t_ref[...] = p_ref[...] * ns


_scale_norms = pl.pallas_call(
    _scale_norms_body,
    out_shape=(jax.ShapeDtypeStruct((N_PAD, D), jnp.float32),
               jax.ShapeDtypeStruct((N_PAD, 2), jnp.float32)),
)


def _mid_body(q_ref, n_ref, b1_ref, w2_ref, o_ref):
    agg = q_ref[0] + q_ref[1]
    h = jnp.maximum(agg * n_ref[:, 1:2] + b1_ref[...], 0.0)
    rid = lax.broadcasted_iota(jnp.int32, (N_PAD, 1), 0)
    h = jnp.where(rid < N, h, 0.0)
    t = jnp.dot(h, w2_ref[...], preferred_element_type=jnp.float32)
    o_ref[...] = t * n_ref[:, 0:1]


_mid = pl.pallas_call(
    _mid_body, out_shape=jax.ShapeDtypeStruct((N_PAD, D), jnp.float32)
)


def _final_body(q_ref, n_ref, b2_ref, o_ref):
    agg = q_ref[0] + q_ref[1]
    o_ref[...] = (agg * n_ref[:, 1:2] + b2_ref[...])[:N]


_final = pl.pallas_call(
    _final_body, out_shape=jax.ShapeDtypeStruct((N, D), jnp.float32)
)


def kernel(features, edge_index, W1, b1, W2, b2):
    ei = edge_index.astype(jnp.int32)
    src3 = ei[0].reshape(NW, CH, K)
    dst3 = ei[1].reshape(NW, CH, K)
    xp = jnp.zeros((N_PAD, D), jnp.float32).at[:N, :].set(features)
    ones_rows = jnp.ones((K,), jnp.float32)
    zero_rows = jnp.zeros((RPT, D), jnp.float32)
    zero_rows_1d = jnp.zeros((RPT,), jnp.float32)

    degp = _degrees(src3, dst3, ones_rows, zero_rows_1d)
    p1 = _matmul(xp, W1)
    t1, norm_t = _scale_norms(p1, degp.reshape(4, N_PAD, 1))
    q1 = _edge_pass(t1, src3, dst3, zero_rows).reshape(2, N_PAD, D)
    t2 = _mid(q1, norm_t, b1.reshape(1, D), W2)
    q2 = _edge_pass(t2, src3, dst3, zero_rows).reshape(2, N_PAD, D)
    return _final(q2, norm_t, b2.reshape(1, D))


# speedup vs baseline: 1.0004x; 1.0004x over previous
import functools

import jax
import jax.numpy as jnp
from jax import lax
from jax.experimental import pallas as pl
from jax.experimental.pallas import tpu as pltpu
from jax.experimental.pallas import tpu_sc as plsc

N = 10000
D = 128
E = 320000
NW = 32
K = 80
CH = 128
E_PAD = NW * CH * K
N_PAD = 10240
RPT = N_PAD // 16

_mesh = plsc.VectorSubcoreMesh(core_axis_name="c", subcore_axis_name="s")


@functools.partial(
    pl.kernel,
    out_type=jax.ShapeDtypeStruct((4 * N_PAD,), jnp.float32),
    mesh=_mesh,
    scratch_types=[
        pltpu.VMEM((CH, K), jnp.int32),
        pltpu.VMEM((CH, K), jnp.int32),
        pltpu.VMEM((K,), jnp.float32),
        pltpu.VMEM_SHARED((N_PAD,), jnp.float32),
        pltpu.VMEM_SHARED((N_PAD,), jnp.float32),
        pltpu.SemaphoreType.DMA,
        pltpu.SemaphoreType.DMA,
    ],
)
def _degrees(src_hbm, dst_hbm, ones_hbm, zeros_hbm, out_hbm,
             src_v, dst_v, ones_v, acc_s, acc_d, sem_a, sem_b):
    c = lax.axis_index("c")
    s = lax.axis_index("s")
    w = c * 16 + s
    pltpu.sync_copy(src_hbm.at[w], src_v)
    pltpu.sync_copy(dst_hbm.at[w], dst_v)
    pltpu.sync_copy(ones_hbm, ones_v)
    pltpu.sync_copy(zeros_hbm, acc_s.at[pl.ds(s * RPT, RPT)])
    pltpu.sync_copy(zeros_hbm, acc_d.at[pl.ds(s * RPT, RPT)])
    plsc.subcore_barrier()

    DGRP = 8

    def gbody(g, _):
        cps = []
        for b in range(DGRP):
            j = g * DGRP + b
            cps.append(pltpu.async_copy(ones_v, acc_s.at[src_v.at[j]], sem_a,
                                        add=True))
            cps.append(pltpu.async_copy(ones_v, acc_d.at[dst_v.at[j]], sem_b,
                                        add=True))
        for cp in cps:
            cp.wait()
        return 0

    lax.fori_loop(0, CH // DGRP, gbody, 0)
    plsc.subcore_barrier()
    base = c * 2 * N_PAD
    pltpu.sync_copy(acc_s.at[pl.ds(s * RPT, RPT)],
                    out_hbm.at[pl.ds(base + s * RPT, RPT)])
    pltpu.sync_copy(acc_d.at[pl.ds(s * RPT, RPT)],
                    out_hbm.at[pl.ds(base + N_PAD + s * RPT, RPT)])


GRP = 16
NG = CH // GRP
NBUF = 4


@functools.partial(
    pl.kernel,
    out_type=jax.ShapeDtypeStruct((2 * N_PAD, D), jnp.float32),
    mesh=_mesh,
    scratch_types=[
        pltpu.VMEM((2, GRP, K), jnp.int32),
        pltpu.VMEM((2, GRP, K), jnp.int32),
        pltpu.VMEM((NBUF, K, D), jnp.float32),
        pltpu.VMEM_SHARED((N_PAD, D), jnp.float32),
        pltpu.SemaphoreType.DMA,
        pltpu.SemaphoreType.DMA,
        pltpu.SemaphoreType.DMA,
        pltpu.SemaphoreType.DMA,
        pltpu.SemaphoreType.DMA,
        pltpu.SemaphoreType.DMA,
        pltpu.SemaphoreType.DMA,
        pltpu.SemaphoreType.DMA,
        pltpu.SemaphoreType.DMA,
        pltpu.SemaphoreType.DMA,
    ],
)
def _edge_pass(t_hbm, src_hbm, dst_hbm, zeros_hbm, out_hbm,
               srcb, dstb, rows, acc,
               gsem0, gsem1, gsem2, gsem3,
               ssem0, ssem1, ssem2, ssem3,
               isem0, isem1):
    c = lax.axis_index("c")
    s = lax.axis_index("s")
    w = c * 16 + s
    my_src = src_hbm.at[w]
    my_dst = dst_hbm.at[w]
    pltpu.sync_copy(my_src.at[pl.ds(0, GRP)], srcb.at[0])
    pltpu.sync_copy(my_dst.at[pl.ds(0, GRP)], dstb.at[0])
    pltpu.sync_copy(zeros_hbm, acc.at[pl.ds(s * RPT, RPT)])
    plsc.subcore_barrier()

    gs = (gsem0, gsem1, gsem2, gsem3)
    ss = (ssem0, ssem1, ssem2, ssem3)
    ip = None
    if NG > 1:
        ip = (pltpu.async_copy(my_src.at[pl.ds(GRP, GRP)], srcb.at[1], isem0),
              pltpu.async_copy(my_dst.at[pl.ds(GRP, GRP)], dstb.at[1], isem1))

    pend_g = {}
    pend_s = {}
    waited_s = set()
    staged = 1

    for cn in range(min(NBUF, CH)):
        b = cn % NBUF
        pend_g[cn] = pltpu.async_copy(
            t_hbm.at[srcb.at[0].at[cn]], rows.at[b], gs[b])

    for cn in range(CH):
        b = cn % NBUF
        pend_g[cn].wait()
        nc = cn + NBUF - 1
        if cn >= 1 and nc < CH:
            g3 = nc // GRP
            if g3 >= staged:
                for cp in ip:
                    cp.wait()
                staged += 1
            pb = (cn - 1) % NBUF
            pend_s[cn - 1].wait()
            waited_s.add(cn - 1)
            pend_g[nc] = pltpu.async_copy(
                t_hbm.at[srcb.at[g3 % 2].at[nc % GRP]], rows.at[pb], gs[pb])
        pend_s[cn] = pltpu.async_copy(
            rows.at[b], acc.at[dstb.at[(cn // GRP) % 2].at[cn % GRP]],
            ss[b], add=True)
        if cn % GRP == 0 and cn > 0:
            gprev = cn // GRP - 1
            if gprev + 2 < NG:
                ip = (pltpu.async_copy(
                          my_src.at[pl.ds((gprev + 2) * GRP, GRP)],
                          srcb.at[gprev % 2], isem0),
                      pltpu.async_copy(
                          my_dst.at[pl.ds((gprev + 2) * GRP, GRP)],
                          dstb.at[gprev % 2], isem1))

    for cn in range(CH):
        if cn not in waited_s:
            pend_s[cn].wait()

    plsc.subcore_barrier()
    pltpu.sync_copy(
        acc.at[pl.ds(s * RPT, RPT)],
        out_hbm.at[pl.ds(c * N_PAD + s * RPT, RPT)],
    )


def _mm_body(x_ref, w_ref, o_ref):
    o_ref[...] = jnp.dot(x_ref[...], w_ref[...],
                         preferred_element_type=jnp.float32)


_matmul = pl.pallas_call(
    _mm_body, out_shape=jax.ShapeDtypeStruct((N_PAD, D), jnp.float32)
)


def _scale_norms_body(p_ref, degp_ref, t_ref, n_ref):
    x = degp_ref[...]
    ns = lax.rsqrt(jnp.maximum(x[0] + x[2], 1.0))
    nd = lax.rsqrt(jnp.maximum(x[1] + x[3], 1.0))
    n_ref[:, 0:1] = ns
    n_ref[:, 1:2] = nd
    t_ref[...] = p_ref[...] * ns


_scale_norms = pl.pallas_call(
    _scale_norms_body,
    out_shape=(jax.ShapeDtypeStruct((N_PAD, D), jnp.float32),
               jax.ShapeDtypeStruct((N_PAD, 2), jnp.float32)),
)


def _mid_body(q_ref, n_ref, b1_ref, w2_ref, o_ref):
    agg = q_ref[0] + q_ref[1]
    h = jnp.maximum(agg * n_ref[:, 1:2] + b1_ref[...], 0.0)
    rid = lax.broadcasted_iota(jnp.int32, (N_PAD, 1), 0)
    h = jnp.where(rid < N, h, 0.0)
    t = jnp.dot(h, w2_ref[...], preferred_element_type=jnp.float32)
    o_ref[...] = t * n_ref[:, 0:1]


_mid = pl.pallas_call(
    _mid_body, out_shape=jax.ShapeDtypeStruct((N_PAD, D), jnp.float32)
)


def _final_body(q_ref, n_ref, b2_ref, o_ref):
    agg = q_ref[0] + q_ref[1]
    o_ref[...] = (agg * n_ref[:, 1:2] + b2_ref[...])[:N]


_final = pl.pallas_call(
    _final_body, out_shape=jax.ShapeDtypeStruct((N, D), jnp.float32)
)


def kernel(features, edge_index, W1, b1, W2, b2):
    ei = edge_index.astype(jnp.int32)
    pad_idx = N + jnp.arange(E_PAD - E, dtype=jnp.int32) % (N_PAD - N)
    pad = jnp.stack([pad_idx, pad_idx])
    eip = jnp.concatenate([ei, pad], axis=1)
    src3 = eip[0].reshape(NW, CH, K)
    dst3 = eip[1].reshape(NW, CH, K)
    xp = jnp.zeros((N_PAD, D), jnp.float32).at[:N, :].set(features)
    ones_rows = jnp.ones((K,), jnp.float32)
    zero_rows = jnp.zeros((RPT, D), jnp.float32)
    zero_rows_1d = jnp.zeros((RPT,), jnp.float32)

    degp = _degrees(src3, dst3, ones_rows, zero_rows_1d)
    p1 = _matmul(xp, W1)
    t1, norm_t = _scale_norms(p1, degp.reshape(4, N_PAD, 1))
    q1 = _edge_pass(t1, src3, dst3, zero_rows).reshape(2, N_PAD, D)
    t2 = _mid(q1, norm_t, b1.reshape(1, D), W2)
    q2 = _edge_pass(t2, src3, dst3, zero_rows).reshape(2, N_PAD, D)
    return _final(q2, norm_t, b2.reshape(1, D))


# speedup vs baseline: 1.0981x; 1.0977x over previous
import functools

import jax
import jax.numpy as jnp
from jax import lax
from jax.experimental import pallas as pl
from jax.experimental.pallas import tpu as pltpu
from jax.experimental.pallas import tpu_sc as plsc

N = 10000
D = 128
E = 320000
NW = 32
K = 80
CH = 128
E_PAD = NW * CH * K
N_PAD = 10240
RPT = N_PAD // 16

_mesh = plsc.VectorSubcoreMesh(core_axis_name="c", subcore_axis_name="s")


@functools.partial(
    pl.kernel,
    out_type=jax.ShapeDtypeStruct((4 * N_PAD,), jnp.float32),
    mesh=_mesh,
    scratch_types=[
        pltpu.VMEM((CH, K), jnp.int32),
        pltpu.VMEM((CH, K), jnp.int32),
        pltpu.VMEM((K,), jnp.float32),
        pltpu.VMEM_SHARED((N_PAD,), jnp.float32),
        pltpu.VMEM_SHARED((N_PAD,), jnp.float32),
        pltpu.SemaphoreType.DMA,
        pltpu.SemaphoreType.DMA,
    ],
)
def _degrees(src_hbm, dst_hbm, ones_hbm, zeros_hbm, out_hbm,
             src_v, dst_v, ones_v, acc_s, acc_d, sem_a, sem_b):
    c = lax.axis_index("c")
    s = lax.axis_index("s")
    w = c * 16 + s
    pltpu.sync_copy(src_hbm.at[w], src_v)
    pltpu.sync_copy(dst_hbm.at[w], dst_v)
    pltpu.sync_copy(ones_hbm, ones_v)
    pltpu.sync_copy(zeros_hbm, acc_s.at[pl.ds(s * RPT, RPT)])
    pltpu.sync_copy(zeros_hbm, acc_d.at[pl.ds(s * RPT, RPT)])
    plsc.subcore_barrier()

    DGRP = 8

    def gbody(g, _):
        cps = []
        for b in range(DGRP):
            j = g * DGRP + b
            cps.append(pltpu.async_copy(ones_v, acc_s.at[src_v.at[j]], sem_a,
                                        add=True))
            cps.append(pltpu.async_copy(ones_v, acc_d.at[dst_v.at[j]], sem_b,
                                        add=True))
        for cp in cps:
            cp.wait()
        return 0

    lax.fori_loop(0, CH // DGRP, gbody, 0)
    plsc.subcore_barrier()
    base = c * 2 * N_PAD
    pltpu.sync_copy(acc_s.at[pl.ds(s * RPT, RPT)],
                    out_hbm.at[pl.ds(base + s * RPT, RPT)])
    pltpu.sync_copy(acc_d.at[pl.ds(s * RPT, RPT)],
                    out_hbm.at[pl.ds(base + N_PAD + s * RPT, RPT)])


GRP = 16
NG = CH // GRP
NBUF = 4


@functools.partial(
    pl.kernel,
    out_type=jax.ShapeDtypeStruct((2 * N_PAD, D), jnp.float32),
    mesh=_mesh,
    scratch_types=[
        pltpu.VMEM((2, GRP, K), jnp.int32),
        pltpu.VMEM((2, GRP, K), jnp.int32),
        pltpu.VMEM((NBUF, K, D), jnp.float32),
        pltpu.VMEM_SHARED((N_PAD, D), jnp.float32),
        pltpu.SemaphoreType.DMA,
        pltpu.SemaphoreType.DMA,
        pltpu.SemaphoreType.DMA,
        pltpu.SemaphoreType.DMA,
        pltpu.SemaphoreType.DMA,
        pltpu.SemaphoreType.DMA,
        pltpu.SemaphoreType.DMA,
        pltpu.SemaphoreType.DMA,
        pltpu.SemaphoreType.DMA,
        pltpu.SemaphoreType.DMA,
    ],
)
def _edge_pass(t_hbm, src_hbm, dst_hbm, zeros_hbm, out_hbm,
               srcb, dstb, rows, acc,
               gsem0, gsem1, gsem2, gsem3,
               ssem0, ssem1, ssem2, ssem3,
               isem0, isem1):
    c = lax.axis_index("c")
    s = lax.axis_index("s")
    w = c * 16 + s
    my_src = src_hbm.at[w]
    my_dst = dst_hbm.at[w]
    pltpu.sync_copy(my_src.at[pl.ds(0, GRP)], srcb.at[0])
    pltpu.sync_copy(my_dst.at[pl.ds(0, GRP)], dstb.at[0])
    pltpu.sync_copy(zeros_hbm, acc.at[pl.ds(s * RPT, RPT)])
    plsc.subcore_barrier()

    gs = (gsem0, gsem1, gsem2, gsem3)
    ss = (ssem0, ssem1, ssem2, ssem3)
    ip = None
    if NG > 1:
        ip = (pltpu.async_copy(my_src.at[pl.ds(GRP, GRP)], srcb.at[1], isem0),
              pltpu.async_copy(my_dst.at[pl.ds(GRP, GRP)], dstb.at[1], isem1))

    pend_g = {}
    pend_s = {}
    waited_s = set()
    staged = 1

    for cn in range(min(NBUF, CH)):
        b = cn % NBUF
        pend_g[cn] = pltpu.async_copy(
            t_hbm.at[srcb.at[0].at[cn]], rows.at[b], gs[b])

    for cn in range(CH):
        b = cn % NBUF
        pend_g[cn].wait()
        nc = cn + NBUF - 1
        if cn >= 1 and nc < CH:
            g3 = nc // GRP
            if g3 >= staged:
                for cp in ip:
                    cp.wait()
                staged += 1
            pb = (cn - 1) % NBUF
            pend_s[cn - 1].wait()
            waited_s.add(cn - 1)
            pend_g[nc] = pltpu.async_copy(
                t_hbm.at[srcb.at[g3 % 2].at[nc % GRP]], rows.at[pb], gs[pb])
        pend_s[cn] = pltpu.async_copy(
            rows.at[b], acc.at[dstb.at[(cn // GRP) % 2].at[cn % GRP]],
            ss[b], add=True)
        if cn % GRP == 0 and cn > 0:
            gprev = cn // GRP - 1
            if gprev + 2 < NG:
                ip = (pltpu.async_copy(
                          my_src.at[pl.ds((gprev + 2) * GRP, GRP)],
                          srcb.at[gprev % 2], isem0),
                      pltpu.async_copy(
                          my_dst.at[pl.ds((gprev + 2) * GRP, GRP)],
                          dstb.at[gprev % 2], isem1))

    for cn in range(CH):
        if cn not in waited_s:
            pend_s[cn].wait()

    plsc.subcore_barrier()
    pltpu.sync_copy(
        acc.at[pl.ds(s * RPT, RPT)],
        out_hbm.at[pl.ds(c * N_PAD + s * RPT, RPT)],
    )


def _mm_body(x_ref, w_ref, o_ref):
    o_ref[0:N, :] = jnp.dot(x_ref[...], w_ref[...],
                            preferred_element_type=jnp.float32)


_matmul = pl.pallas_call(
    _mm_body, out_shape=jax.ShapeDtypeStruct((N_PAD, D), jnp.float32)
)


def _scale_norms_body(p_ref, degp_ref, t_ref, n_ref):
    x = degp_ref[...]
    ns = lax.rsqrt(jnp.maximum(x[:, 0:1] + x[:, 2:3], 1.0))
    nd = lax.rsqrt(jnp.maximum(x[:, 1:2] + x[:, 3:4], 1.0))
    n_ref[:, 0:1] = ns
    n_ref[:, 1:2] = nd
    t_ref[...] = p_ref[...] * ns


_scale_norms = pl.pallas_call(
    _scale_norms_body,
    out_shape=(jax.ShapeDtypeStruct((N_PAD, D), jnp.float32),
               jax.ShapeDtypeStruct((N_PAD, 2), jnp.float32)),
)


def _mid_body(q_ref, n_ref, b1_ref, w2_ref, o_ref):
    agg = q_ref[0] + q_ref[1]
    h = jnp.maximum(agg * n_ref[:, 1:2] + b1_ref[...], 0.0)
    rid = lax.broadcasted_iota(jnp.int32, (N_PAD, 1), 0)
    h = jnp.where(rid < N, h, 0.0)
    t = jnp.dot(h, w2_ref[...], preferred_element_type=jnp.float32)
    o_ref[...] = t * n_ref[:, 0:1]


_mid = pl.pallas_call(
    _mid_body, out_shape=jax.ShapeDtypeStruct((N_PAD, D), jnp.float32)
)


def _final_body(q_ref, n_ref, b2_ref, o_ref):
    agg = q_ref[0] + q_ref[1]
    o_ref[...] = (agg * n_ref[:, 1:2] + b2_ref[...])[:N]


_final = pl.pallas_call(
    _final_body, out_shape=jax.ShapeDtypeStruct((N, D), jnp.float32)
)


def kernel(features, edge_index, W1, b1, W2, b2):
    ei = edge_index.astype(jnp.int32)
    pad_idx = N + jnp.arange(E_PAD - E, dtype=jnp.int32) % (N_PAD - N)
    pad = jnp.stack([pad_idx, pad_idx])
    eip = jnp.concatenate([ei, pad], axis=1)
    src3 = eip[0].reshape(NW, CH, K)
    dst3 = eip[1].reshape(NW, CH, K)
    ones_rows = jnp.ones((K,), jnp.float32)
    zero_rows = jnp.zeros((RPT, D), jnp.float32)
    zero_rows_1d = jnp.zeros((RPT,), jnp.float32)

    degp = _degrees(src3, dst3, ones_rows, zero_rows_1d)
    p1 = _matmul(features, W1)
    t1, norm_t = _scale_norms(p1, degp.reshape(4, N_PAD).T)
    q1 = _edge_pass(t1, src3, dst3, zero_rows).reshape(2, N_PAD, D)
    t2 = _mid(q1, norm_t, b1.reshape(1, D), W2)
    q2 = _edge_pass(t2, src3, dst3, zero_rows).reshape(2, N_PAD, D)
    return _final(q2, norm_t, b2.reshape(1, D))
